# baseline (speedup 1.0000x reference)
import functools

import jax
import jax.numpy as jnp
from jax.experimental import pallas as pl
from jax.experimental.pallas import tpu as pltpu

_BLK = 256
_NBUF = 4


def _gcn_kernel(inter_ref, adj1_hbm, adj2_hbm, de_ref, me_ref, pe_ref,
                o1_ref, o2_ref, buf1, buf2, sem1, sem2, *, blk, nbuf, half):
    n = o1_ref.shape[0]
    nchunks = n // blk
    d = de_ref.shape[0]

    def cp(j, slot, hbm, buf, sem):
        return pltpu.make_async_copy(
            hbm.at[pl.ds(j * blk, blk), :], buf.at[slot], sem.at[slot])

    for s in range(nbuf - 1):
        cp(s, s, adj1_hbm, buf1, sem1).start()
        cp(s, s, adj2_hbm, buf2, sem2).start()

    de = de_ref[...]
    me = me_ref[...]
    pe = pe_ref[...]
    w = inter_ref[0]

    for j in range(nchunks):
        ahead = j + nbuf - 1
        if ahead < nchunks:
            cp(ahead, ahead % nbuf, adj1_hbm, buf1, sem1).start()
            cp(ahead, ahead % nbuf, adj2_hbm, buf2, sem2).start()
        slot = j % nbuf
        cp(j, slot, adj1_hbm, buf1, sem1).wait()
        cp(j, slot, adj2_hbm, buf2, sem2).wait()
        a1 = buf1[slot]
        a2 = buf2[slot]
        y1 = (jnp.dot(a1[:, :d], de, preferred_element_type=jnp.float32) +
              jnp.dot(a1[:, d:], me, preferred_element_type=jnp.float32))
        y2 = (jnp.dot(a2[:, :d], pe, preferred_element_type=jnp.float32) +
              jnp.dot(a2[:, d:], me, preferred_element_type=jnp.float32))
        t1 = 4.0 * jnp.maximum(y1, 0.0)
        t2 = 4.0 * jnp.maximum(y2, 0.0)
        rows = pl.ds(j * blk, blk)
        o1_ref[rows, :] = t1
        if j < half:
            o2_ref[rows, :] = t2
        else:
            o2_ref[rows, :] = w * t1 + (1.0 - w) * t2


def kernel(adj1, adj2, dEmbed, mEmbed, pEmbed, inter):
    n = adj1.shape[0]
    d = dEmbed.shape[0]
    m = mEmbed.shape[0]
    p = pEmbed.shape[0]
    f = dEmbed.shape[1]
    blk = _BLK
    nbuf = _NBUF
    half = d // blk

    o1, o2 = pl.pallas_call(
        functools.partial(_gcn_kernel, blk=blk, nbuf=nbuf, half=half),
        grid=(1,),
        in_specs=[
            pl.BlockSpec(memory_space=pltpu.SMEM),
            pl.BlockSpec(memory_space=pl.ANY),
            pl.BlockSpec(memory_space=pl.ANY),
            pl.BlockSpec((d, f), lambda i: (0, 0)),
            pl.BlockSpec((m, f), lambda i: (0, 0)),
            pl.BlockSpec((p, f), lambda i: (0, 0)),
        ],
        out_specs=[
            pl.BlockSpec((n, f), lambda i: (0, 0)),
            pl.BlockSpec((n, f), lambda i: (0, 0)),
        ],
        out_shape=[
            jax.ShapeDtypeStruct((n, f), jnp.float32),
            jax.ShapeDtypeStruct((n, f), jnp.float32),
        ],
        scratch_shapes=[
            pltpu.VMEM((nbuf, blk, n), jnp.float32),
            pltpu.VMEM((nbuf, blk, n), jnp.float32),
            pltpu.SemaphoreType.DMA((nbuf,)),
            pltpu.SemaphoreType.DMA((nbuf,)),
        ],
    )(inter, adj1, adj2, dEmbed, mEmbed, pEmbed)
    return (o2[p:], o1[:d], o2[:p])


# speedup vs baseline: 1.0427x; 1.0427x over previous
import functools

import jax
import jax.numpy as jnp
from jax.experimental import pallas as pl
from jax.experimental.pallas import tpu as pltpu

_BLK = 256


def _gcn_kernel(inter_ref, adj1_ref, adj2_hbm, de_ref, me_ref, pe_ref,
                o1_ref, o2_ref, buf2, sem2, *, blk, half, nchunks):
    i = pl.program_id(0)
    d = de_ref.shape[0]

    def cp(j, slot):
        return pltpu.make_async_copy(
            adj2_hbm.at[pl.ds(j * blk, blk), :], buf2.at[slot], sem2.at[slot])

    @pl.when(i == 0)
    def _():
        cp(0, 0).start()

    @pl.when(i + 1 < nchunks)
    def _():
        nxt = i + 1
        cp(nxt, jax.lax.rem(nxt, 2)).start()

    me = me_ref[...]
    y1 = (jnp.dot(adj1_ref[:, :d], de_ref[...],
                  preferred_element_type=jnp.float32) +
          jnp.dot(adj1_ref[:, d:], me, preferred_element_type=jnp.float32))

    slot = jax.lax.rem(i, 2)
    cp(i, slot).wait()
    a2 = buf2[slot]
    y2 = (jnp.dot(a2[:, :d], pe_ref[...],
                  preferred_element_type=jnp.float32) +
          jnp.dot(a2[:, d:], me, preferred_element_type=jnp.float32))
    t1 = 4.0 * jnp.maximum(y1, 0.0)
    t2 = 4.0 * jnp.maximum(y2, 0.0)
    o1_ref[...] = t1

    @pl.when(i < half)
    def _():
        o2_ref[...] = t2

    @pl.when(i >= half)
    def _():
        w = inter_ref[0]
        o2_ref[...] = w * t1 + (1.0 - w) * t2


def kernel(adj1, adj2, dEmbed, mEmbed, pEmbed, inter):
    n = adj1.shape[0]
    d = dEmbed.shape[0]
    m = mEmbed.shape[0]
    p = pEmbed.shape[0]
    f = dEmbed.shape[1]
    blk = _BLK
    grid = n // blk
    half = d // blk

    o1, o2 = pl.pallas_call(
        functools.partial(_gcn_kernel, blk=blk, half=half, nchunks=grid),
        grid=(grid,),
        in_specs=[
            pl.BlockSpec(memory_space=pltpu.SMEM),
            pl.BlockSpec((blk, n), lambda i: (i, 0)),
            pl.BlockSpec(memory_space=pl.ANY),
            pl.BlockSpec((d, f), lambda i: (0, 0)),
            pl.BlockSpec((m, f), lambda i: (0, 0)),
            pl.BlockSpec((p, f), lambda i: (0, 0)),
        ],
        out_specs=[
            pl.BlockSpec((blk, f), lambda i: (i, 0)),
            pl.BlockSpec((blk, f), lambda i: (i, 0)),
        ],
        out_shape=[
            jax.ShapeDtypeStruct((n, f), jnp.float32),
            jax.ShapeDtypeStruct((n, f), jnp.float32),
        ],
        scratch_shapes=[
            pltpu.VMEM((2, blk, n), jnp.float32),
            pltpu.SemaphoreType.DMA((2,)),
        ],
    )(inter, adj1, adj2, dEmbed, mEmbed, pEmbed)
    return (o2[p:], o1[:d], o2[:p])


# speedup vs baseline: 1.0858x; 1.0414x over previous
import functools

import jax
import jax.numpy as jnp
from jax.experimental import pallas as pl
from jax.experimental.pallas import tpu as pltpu

_BLK = 256


def _gcn_kernel(inter_ref, adj1_ref, adj2_ref, de_ref, me_ref, pe_ref,
                o1_ref, o2_ref, *, half):
    i = pl.program_id(0)
    d = de_ref.shape[0]
    me = me_ref[...]
    y1 = (jnp.dot(adj1_ref[:, :d], de_ref[...],
                  preferred_element_type=jnp.float32) +
          jnp.dot(adj1_ref[:, d:], me, preferred_element_type=jnp.float32))
    y2 = (jnp.dot(adj2_ref[:, :d], pe_ref[...],
                  preferred_element_type=jnp.float32) +
          jnp.dot(adj2_ref[:, d:], me, preferred_element_type=jnp.float32))
    t1 = 4.0 * jnp.maximum(y1, 0.0)
    t2 = 4.0 * jnp.maximum(y2, 0.0)
    o1_ref[...] = t1

    @pl.when(i < half)
    def _():
        o2_ref[...] = t2

    @pl.when(i >= half)
    def _():
        w = inter_ref[0]
        o2_ref[...] = w * t1 + (1.0 - w) * t2


def kernel(adj1, adj2, dEmbed, mEmbed, pEmbed, inter):
    n = adj1.shape[0]
    d = dEmbed.shape[0]
    m = mEmbed.shape[0]
    p = pEmbed.shape[0]
    f = dEmbed.shape[1]
    blk = _BLK
    grid = n // blk
    half = d // blk

    o1, o2 = pl.pallas_call(
        functools.partial(_gcn_kernel, half=half),
        grid=(grid,),
        in_specs=[
            pl.BlockSpec(memory_space=pltpu.SMEM),
            pl.BlockSpec((blk, n), lambda i: (i, 0)),
            pl.BlockSpec((blk, n), lambda i: (i, 0)),
            pl.BlockSpec((d, f), lambda i: (0, 0)),
            pl.BlockSpec((m, f), lambda i: (0, 0)),
            pl.BlockSpec((p, f), lambda i: (0, 0)),
        ],
        out_specs=[
            pl.BlockSpec((blk, f), lambda i: (i, 0)),
            pl.BlockSpec((blk, f), lambda i: (i, 0)),
        ],
        out_shape=[
            jax.ShapeDtypeStruct((n, f), jnp.float32),
            jax.ShapeDtypeStruct((n, f), jnp.float32),
        ],
        compiler_params=pltpu.CompilerParams(
            dimension_semantics=("parallel",),
        ),
    )(inter, adj1, adj2, dEmbed, mEmbed, pEmbed)
    return (o2[p:], o1[:d], o2[:p])
